# Initial kernel scaffold; baseline (speedup 1.0000x reference)
import jax
import jax.numpy as jnp
from jax.experimental import pallas as pl


def kernel(pos, edge_index, bond_mask, z, batch, bond_table, atom_table, W_init, Wq, Wk, Wvs, Wos, Wsv, Wgv, Wov, Wgate, Wh, Wout):
    raise NotImplementedError("write your pallas kernel here")


# speedup vs baseline: 18.8473x; 18.8473x over previous
import functools

import numpy as np
import jax
import jax.numpy as jnp
from jax import lax
from jax.experimental import pallas as pl
from jax.experimental.pallas import tpu as pltpu
from jax.experimental.pallas import tpu_sc as plsc

F32 = jnp.float32
I32 = jnp.int32

NC = 2
NS = 16
NW = NC * NS

_N = 50000
_E = 800000
_NS_DIM = 64
_H = 4
_DH = 16
_RADIAL = 8
_MAX_R = 5.0
_NG = 32

_N_PAD = 51200

_HIGH = lax.Precision.HIGHEST


def _dot(a, b):
    return jnp.dot(a, b, preferred_element_type=F32, precision=_HIGH)


def _silu(x):
    return x * jax.nn.sigmoid(x)


def _mesh():
    return plsc.VectorSubcoreMesh(
        core_axis_name="c", subcore_axis_name="s", num_cores=NC, num_subcores=NS
    )


_GATHER_CACHE = {}


def _make_gather(m, t_rows, d, chunk):
    per_w = m // NW
    assert m % NW == 0 and per_w % chunk == 0 and chunk % 8 == 0 and d % 16 == 0

    @functools.partial(
        pl.kernel,
        out_type=jax.ShapeDtypeStruct((m, d), F32),
        mesh=_mesh(),
        scratch_types=[
            pltpu.VMEM((chunk,), I32),
            pltpu.VMEM((chunk, d), F32),
            pltpu.SemaphoreType.DMA,
        ],
        compiler_params=pltpu.CompilerParams(use_tc_tiling_on_sc=False),
    )
    def gather_k(tab, idx, out, idx_v, rows_v, sem):
        wid = lax.axis_index("s") * NC + lax.axis_index("c")
        base_w = wid * per_w
        for off in range(0, per_w, chunk):
            b = base_w + off
            pltpu.sync_copy(idx.at[pl.ds(b, chunk)], idx_v)
            pltpu.async_copy(tab.at[idx_v], rows_v, sem).wait()
            pltpu.sync_copy(rows_v, out.at[pl.ds(b, chunk)])

    return gather_k


def _sc_gather(table, idx, chunk=1000):
    m = idx.shape[0]
    t_rows, d = table.shape
    key = (m, t_rows, d, chunk)
    if key not in _GATHER_CACHE:
        _GATHER_CACHE[key] = _make_gather(m, t_rows, d, chunk)
    return _GATHER_CACHE[key](table, idx)


_SCATTER_CACHE = {}


def _make_scatter(m, d, n_out, chunk=400):
    half = (n_out + 1) // 2
    half_pad = ((half + 1 + NS - 1) // NS) * NS
    rpt = half_pad // NS
    per_tile = m // NS
    assert m % NS == 0 and per_tile % chunk == 0 and chunk % 16 == 0
    assert d % 16 == 0
    zero_plan = [(o, min(512, rpt - o)) for o in range(0, rpt, 512)]

    @functools.partial(
        pl.kernel,
        out_type=jax.ShapeDtypeStruct((NC, half_pad, d), F32),
        mesh=_mesh(),
        scratch_types=[
            pltpu.VMEM((chunk,), I32),
            pltpu.VMEM((chunk,), I32),
            pltpu.VMEM((chunk, d), F32),
            pltpu.VMEM_SHARED((half_pad, d), F32),
        ],
        compiler_params=pltpu.CompilerParams(use_tc_tiling_on_sc=False),
    )
    def scatter_k(vals, idx, zeros, out, idx_v, lidx_v, val_v, acc):
        sc = lax.axis_index("c")
        tid = lax.axis_index("s")
        base_node = sc * half
        tile_r0 = tid * rpt
        for ro, rs in zero_plan:
            pltpu.sync_copy(
                zeros.at[pl.ds(tile_r0 + ro, rs)], acc.at[pl.ds(tile_r0 + ro, rs)]
            )
        plsc.subcore_barrier()
        tb = tid * per_tile
        for off in range(0, per_tile, chunk):
            b = tb + off
            pltpu.sync_copy(idx.at[pl.ds(b, chunk)], idx_v)
            pltpu.sync_copy(vals.at[pl.ds(b, chunk)], val_v)
            for j in range(chunk // 16):
                v = idx_v[pl.ds(j * 16, 16)]
                lv = v - base_node
                ok = (lv >= 0) & (lv < half)
                lidx_v[pl.ds(j * 16, 16)] = jnp.where(ok, lv, half)
            pltpu.sync_copy(val_v, acc.at[lidx_v], add=True)
        plsc.subcore_barrier()
        for ro, rs in zero_plan:
            pltpu.sync_copy(
                acc.at[pl.ds(tile_r0 + ro, rs)],
                out.at[sc, pl.ds(tile_r0 + ro, rs), :],
            )

    return scatter_k, half, half_pad


def _sc_segment_sum(vals, idx, n_out, chunk=400):
    m, d = vals.shape
    key = (m, d, n_out, chunk)
    if key not in _SCATTER_CACHE:
        _SCATTER_CACHE[key] = _make_scatter(m, d, n_out, chunk)
    scatter_k, half, half_pad = _SCATTER_CACHE[key]
    zeros = jnp.zeros((half_pad, d), F32)
    halves = scatter_k(vals, idx, zeros)
    return jnp.concatenate([halves[0, :half], halves[1, :half]], axis=0)[:n_out]


_BE = 3200
_BN = 2000


def _edge_specs(shapes):
    return [pl.BlockSpec((_BE, s), lambda i: (i, 0)) for s in shapes]


def _full_spec(shape):
    return pl.BlockSpec(shape, lambda i: tuple(0 for _ in shape))


def _tc_edge_attr(ps, pd, bm, bond_table_pad):
    step = float(_MAX_R) / (_RADIAL + 1)

    def body(ps_ref, pd_ref, bm_ref, bt_ref, out_ref):
        dvec = ps_ref[...] - pd_ref[...]
        r2 = jnp.sum(dvec * dvec, axis=1, keepdims=True) + 1e-12
        r = jnp.sqrt(r2)
        centers = (
            lax.broadcasted_iota(I32, (1, _RADIAL), 1).astype(F32) + 1.0
        ) * step
        diff = (r - centers) / step
        radial = jnp.exp(-diff * diff) / 1.12
        bt = bt_ref[...]
        bonded = jnp.where(bm_ref[...] == 1, bt[1:2, :], bt[0:1, :])
        out_ref[...] = jnp.concatenate([bonded, radial], axis=1)

    return pl.pallas_call(
        body,
        grid=(_E // _BE,),
        in_specs=_edge_specs([16, 16, 1]) + [_full_spec((8, 8))],
        out_specs=pl.BlockSpec((_BE, 16), lambda i: (i, 0)),
        out_shape=jax.ShapeDtypeStruct((_E, 16), F32),
    )(ps, pd, bm, bond_table_pad)


def _tc_atom_proj(atom_table_pad, w_init):
    def body(a_ref, w_ref, out_ref):
        out_ref[...] = _dot(a_ref[...], w_ref[...])

    return pl.pallas_call(
        body,
        out_shape=jax.ShapeDtypeStruct((104, _NS_DIM), F32),
    )(atom_table_pad, w_init)


def _tc_node_proj(x_s, wq, wks, wvs):
    def body(xs_ref, wq_ref, wks_ref, wvs_ref, q_ref, kx_ref, vs_ref):
        xs = xs_ref[...]
        q_ref[...] = _dot(xs, wq_ref[...])
        kx_ref[...] = _dot(xs, wks_ref[...])
        vs_ref[...] = _dot(xs, wvs_ref[...])

    n = x_s.shape[0]
    sds = jax.ShapeDtypeStruct((n, _NS_DIM), F32)
    return pl.pallas_call(
        body,
        grid=(n // _BN,),
        in_specs=[pl.BlockSpec((_BN, _NS_DIM), lambda i: (i, 0))]
        + [_full_spec((_NS_DIM, _NS_DIM))] * 3,
        out_specs=[pl.BlockSpec((_BN, _NS_DIM), lambda i: (i, 0))] * 3,
        out_shape=[sds, sds, sds],
    )(x_s, wq, wks, wvs)


def _tc_logits_exp(qd, kxs, edge_attr, wke, g64):
    def body(qd_ref, kx_ref, ea_ref, w_ref, g_ref, out_ref):
        k = kx_ref[...] + _dot(ea_ref[...], w_ref[...])
        p = qd_ref[...] * k
        l4 = _dot(p, g_ref[...]) * 0.25
        hmask = jnp.where(lax.broadcasted_iota(I32, (1, 16), 1) < _H, 1.0, 0.0)
        out_ref[...] = jnp.exp(jnp.minimum(l4, 80.0)) * hmask.astype(F32)

    return pl.pallas_call(
        body,
        grid=(_E // _BE,),
        in_specs=_edge_specs([_NS_DIM, _NS_DIM, 16])
        + [_full_spec((16, _NS_DIM)), _full_spec((_NS_DIM, 16))],
        out_specs=pl.BlockSpec((_BE, 16), lambda i: (i, 0)),
        out_shape=jax.ShapeDtypeStruct((_E, 16), F32),
    )(qd, kxs, edge_attr, wke, g64)


def _tc_messages(vss, ex, r16):
    def body(vs_ref, ex_ref, r_ref, out_ref):
        w = _dot(ex_ref[...], r_ref[...])
        out_ref[...] = vs_ref[...] * w

    return pl.pallas_call(
        body,
        grid=(_E // _BE,),
        in_specs=_edge_specs([_NS_DIM, 16]) + [_full_spec((16, _NS_DIM))],
        out_specs=pl.BlockSpec((_BE, _NS_DIM), lambda i: (i, 0)),
        out_shape=jax.ShapeDtypeStruct((_E, _NS_DIM), F32),
    )(vss, ex, r16)


def _tc_node_update(x_s, agg, den, r16, wos):
    def body(xs_ref, agg_ref, den_ref, r_ref, wos_ref, out_ref):
        den_e = _dot(den_ref[...], r_ref[...])
        aggn = agg_ref[...] / (den_e + 1e-30)
        t = xs_ref[...] + _silu(_dot(aggn, wos_ref[...]))
        out_ref[...] = _silu(t)

    n = x_s.shape[0]
    return pl.pallas_call(
        body,
        grid=(n // _BN,),
        in_specs=[
            pl.BlockSpec((_BN, _NS_DIM), lambda i: (i, 0)),
            pl.BlockSpec((_BN, _NS_DIM), lambda i: (i, 0)),
            pl.BlockSpec((_BN, 16), lambda i: (i, 0)),
            _full_spec((16, _NS_DIM)),
            _full_spec((_NS_DIM, _NS_DIM)),
        ],
        out_specs=pl.BlockSpec((_BN, _NS_DIM), lambda i: (i, 0)),
        out_shape=jax.ShapeDtypeStruct((n, _NS_DIM), F32),
    )(x_s, agg, den, r16, wos)


def _tc_head(pooled, wh, wout_row):
    def body(p_ref, wh_ref, wo_ref, out_ref):
        h = _silu(_dot(p_ref[...], wh_ref[...]))
        out_ref[...] = jnp.sum(h * wo_ref[...], axis=1, keepdims=True)

    return pl.pallas_call(
        body,
        out_shape=jax.ShapeDtypeStruct((_NG, 1), F32),
    )(pooled, wh, wout_row)


def kernel(pos, edge_index, bond_mask, z, batch, bond_table, atom_table, W_init,
           Wq, Wk, Wvs, Wos, Wsv, Wgv, Wov, Wgate, Wh, Wout):
    num_layers = Wq.shape[0]

    src = edge_index[0].astype(I32)
    dst = edge_index[1].astype(I32)
    bm = bond_mask.astype(I32).reshape(_E, 1)

    g64 = np.zeros((_NS_DIM, 16), np.float32)
    for h in range(_H):
        g64[h * _DH:(h + 1) * _DH, h] = 1.0
    g64 = jnp.asarray(g64)
    r16 = np.zeros((16, _NS_DIM), np.float32)
    for h in range(_H):
        r16[h, h * _DH:(h + 1) * _DH] = 1.0
    r16 = jnp.asarray(r16)

    pos_pad = jnp.concatenate([pos.astype(F32), jnp.zeros((_N, 13), F32)], axis=1)
    ps = _sc_gather(pos_pad, src)
    pd = _sc_gather(pos_pad, dst)
    bond_table_pad = jnp.concatenate(
        [bond_table.astype(F32), jnp.zeros((6, 8), F32)], axis=0
    )
    edge_attr = _tc_edge_attr(ps, pd, bm, bond_table_pad)

    atom_table_pad = jnp.concatenate(
        [atom_table.astype(F32), jnp.zeros((4, _NS_DIM), F32)], axis=0
    )
    a_proj = _tc_atom_proj(atom_table_pad, W_init.astype(F32))
    z_pad = jnp.pad(z.astype(I32), (0, _N_PAD - _N))
    x_s = _sc_gather(a_proj, z_pad, chunk=800)[:_N]

    for l in range(num_layers):
        wq_l = Wq[l].astype(F32)
        wks_l = Wk[l][:_NS_DIM].astype(F32)
        wke_l = Wk[l][_NS_DIM:].astype(F32)
        wvs_l = Wvs[l].astype(F32)
        wos_l = Wos[l].astype(F32)

        q, kx, vs = _tc_node_proj(x_s, wq_l, wks_l, wvs_l)
        qd = _sc_gather(q, dst)
        kxs = _sc_gather(kx, src)
        ex = _tc_logits_exp(qd, kxs, edge_attr, wke_l, g64)
        den = _sc_segment_sum(ex, dst, _N)
        vss = _sc_gather(vs, src)
        mv = _tc_messages(vss, ex, r16)
        agg = _sc_segment_sum(mv, dst, _N)
        x_s = _tc_node_update(x_s, agg, den, r16, wos_l)

    x_pad = jnp.pad(x_s, ((0, _N_PAD - _N), (0, 0)))
    batch_pad = jnp.pad(batch.astype(I32), (0, _N_PAD - _N), constant_values=_NG)
    pooled = _sc_segment_sum(x_pad, batch_pad, _NG)
    return _tc_head(pooled, Wh.astype(F32), Wout.astype(F32).reshape(1, _NS_DIM))
